# baseline (speedup 1.0000x reference)
import functools

import jax
import jax.numpy as jnp
from jax import lax
from jax.experimental import pallas as pl
from jax.experimental.pallas import tpu as pltpu
from jax.experimental.pallas import tpu_sc as plsc

BATCH = 4096
HIST = 50
EMBED_DIM = 64

NC = 2
NS = 16
NW = NC * NS

BW = BATCH // NW


def _transpose_chunk(c, t):
    iota = lax.iota(jnp.int32, 16)
    q_vec = jnp.bitwise_and(iota, 7)
    s_half = jax.lax.shift_right_logical(iota, 3)
    s_vecs = [s_half + 2 * e16 for e16 in range(4)]

    @pl.loop(0, BW, step=2)
    def _(m):
        for mm in (m, m + 1):
            vals = [c[mm, pl.ds(16 * e16, 16)] for e16 in range(4)]
            mv = jnp.full((16,), mm, jnp.int32)
            for e16 in range(4):
                plsc.store_scatter(t, [s_vecs[e16], q_vec, mv], vals[e16])


def _gather_body(xt_hbm, table_hbm, out_hbm, idx_v, c0, c1, t0, t1,
                 gsem0, gsem1, ssem0, ssem1):
    cid = lax.axis_index("c")
    sid = lax.axis_index("s")
    wid = sid * NC + cid

    pltpu.sync_copy(xt_hbm.at[:, pl.ds(wid * BW, BW)], idx_v)

    pltpu.async_copy(table_hbm.at[idx_v.at[0]], c0, gsem0)

    @pl.loop(0, HIST, step=2)
    def _(h):
        pltpu.make_async_copy(table_hbm.at[idx_v.at[h]], c0, gsem0).wait()
        pltpu.async_copy(table_hbm.at[idx_v.at[h + 1]], c1, gsem1)
        _transpose_chunk(c0, t0)
        pltpu.async_copy(
            t0.at[:, :, :BW], out_hbm.at[pl.ds(h * 8, 8), wid], ssem0
        )
        pltpu.make_async_copy(table_hbm.at[idx_v.at[h + 1]], c1, gsem1).wait()

        @pl.when(h + 2 < HIST)
        def _():
            pltpu.async_copy(table_hbm.at[idx_v.at[h + 2]], c0, gsem0)

        _transpose_chunk(c1, t1)
        pltpu.async_copy(
            t1.at[:, :, :BW], out_hbm.at[pl.ds((h + 1) * 8, 8), wid], ssem1
        )
        pltpu.make_async_copy(
            t0.at[:, :, :BW], out_hbm.at[pl.ds(h * 8, 8), wid], ssem0
        ).wait()
        pltpu.make_async_copy(
            t1.at[:, :, :BW], out_hbm.at[pl.ds((h + 1) * 8, 8), wid], ssem1
        ).wait()


@jax.jit
def _sc_gather(xt, table):
    mesh = plsc.VectorSubcoreMesh(core_axis_name="c", subcore_axis_name="s")
    f = functools.partial(
        pl.kernel,
        out_type=jax.ShapeDtypeStruct((HIST * 8, NW, 8, BW), jnp.float32),
        mesh=mesh,
        scratch_types=[
            pltpu.VMEM((HIST, BW), jnp.int32),
            pltpu.VMEM((BW, 2 * EMBED_DIM), jnp.float32),
            pltpu.VMEM((BW, 2 * EMBED_DIM), jnp.float32),
            pltpu.VMEM((8, 8, BW + 1), jnp.float32),
            pltpu.VMEM((8, 8, BW + 1), jnp.float32),
            pltpu.SemaphoreType.DMA,
            pltpu.SemaphoreType.DMA,
            pltpu.SemaphoreType.DMA,
            pltpu.SemaphoreType.DMA,
        ],
        compiler_params=pltpu.CompilerParams(
            use_tc_tiling_on_sc=False,
            needs_layout_passes=False,
            disable_bounds_checks=True,
        ),
    )(_gather_body)
    return f(xt, table)


def _mask_body(x_ref, o_ref):
    o_ref[...] = x_ref[...] >= 1


@jax.jit
def _tc_mask(x):
    return pl.pallas_call(
        _mask_body,
        out_shape=jax.ShapeDtypeStruct((BATCH, HIST), jnp.bool_),
    )(x)


def kernel(x, weight_matrix):
    table_p = jnp.pad(weight_matrix, ((0, 0), (0, 2 * EMBED_DIM - EMBED_DIM)))
    l4 = _sc_gather(x.T, table_p)
    embeds = (
        l4.reshape(HIST, 8, NW, 8, BW)
        .transpose(2, 4, 0, 1, 3)
        .reshape(BATCH, HIST, EMBED_DIM)
    )
    mask = _tc_mask(x)
    return embeds, mask


# speedup vs baseline: 1.0516x; 1.0516x over previous
import functools

import jax
import jax.numpy as jnp
from jax import lax
from jax.experimental import pallas as pl
from jax.experimental.pallas import tpu as pltpu
from jax.experimental.pallas import tpu_sc as plsc

BATCH = 4096
HIST = 50
EMBED_DIM = 64

NC = 2
NS = 16
NW = NC * NS

BW = BATCH // NW


def _transpose_chunk(c, t):
    iota = lax.iota(jnp.int32, 16)
    q_vec = jnp.bitwise_and(iota, 7)
    s_half = jax.lax.shift_right_logical(iota, 3)
    s_vecs = [s_half + 2 * e16 for e16 in range(4)]

    @pl.loop(0, BW, step=2)
    def _(m):
        for mm in (m, m + 1):
            vals = [c[mm, pl.ds(16 * e16, 16)] for e16 in range(4)]
            mv = jnp.full((16,), mm, jnp.int32)
            for e16 in range(4):
                plsc.store_scatter(t, [s_vecs[e16], q_vec, mv], vals[e16])


def _gather_body(xt_hbm, table_hbm, out_hbm, idx_v, c0, c1, c2, c3, t0, t1,
                 g0, g1, g2, g3, s0, s1):
    cid = lax.axis_index("c")
    sid = lax.axis_index("s")
    wid = sid * NC + cid

    cs = [c0, c1, c2, c3]
    gs = [g0, g1, g2, g3]
    ts = [t0, t1]
    ss = [s0, s1]

    pltpu.sync_copy(xt_hbm.at[:, pl.ds(wid * BW, BW)], idx_v)

    for j in range(3):
        pltpu.async_copy(table_hbm.at[idx_v.at[j]], cs[j], gs[j])

    def sub(h, k, fire=True):
        hk = h + k
        c, gsem = cs[k % 4], gs[k % 4]
        t, ssem = ts[k % 2], ss[k % 2]
        pltpu.make_async_copy(table_hbm.at[idx_v.at[hk]], c, gsem).wait()

        if fire:

            @pl.when(hk + 3 < HIST)
            def _():
                cn = (k + 3) % 4
                pltpu.async_copy(table_hbm.at[idx_v.at[hk + 3]], cs[cn], gs[cn])

        @pl.when(hk >= 2)
        def _():
            pltpu.make_async_copy(
                t.at[:, :, :BW], out_hbm.at[pl.ds((hk - 2) * 8, 8), wid], ssem
            ).wait()

        _transpose_chunk(c, t)
        pltpu.async_copy(
            t.at[:, :, :BW], out_hbm.at[pl.ds(hk * 8, 8), wid], ssem
        )

    @pl.loop(0, HIST - 2, step=4)
    def _(h):
        for k in range(4):
            sub(h, k)

    sub(HIST - 2, 0, fire=False)
    sub(HIST - 2, 1, fire=False)
    for k, hk in ((0, HIST - 2), (1, HIST - 1)):
        pltpu.make_async_copy(
            ts[k].at[:, :, :BW], out_hbm.at[pl.ds(hk * 8, 8), wid], ss[k]
        ).wait()


@jax.jit
def _sc_gather(xt, table):
    mesh = plsc.VectorSubcoreMesh(core_axis_name="c", subcore_axis_name="s")
    f = functools.partial(
        pl.kernel,
        out_type=jax.ShapeDtypeStruct((HIST * 8, NW, 8, BW), jnp.float32),
        mesh=mesh,
        scratch_types=[
            pltpu.VMEM((HIST, BW), jnp.int32),
            pltpu.VMEM((BW, EMBED_DIM), jnp.float32),
            pltpu.VMEM((BW, EMBED_DIM), jnp.float32),
            pltpu.VMEM((BW, EMBED_DIM), jnp.float32),
            pltpu.VMEM((BW, EMBED_DIM), jnp.float32),
            pltpu.VMEM((8, 8, BW + 1), jnp.float32),
            pltpu.VMEM((8, 8, BW + 1), jnp.float32),
            pltpu.SemaphoreType.DMA,
            pltpu.SemaphoreType.DMA,
            pltpu.SemaphoreType.DMA,
            pltpu.SemaphoreType.DMA,
            pltpu.SemaphoreType.DMA,
            pltpu.SemaphoreType.DMA,
        ],
        compiler_params=pltpu.CompilerParams(
            use_tc_tiling_on_sc=False,
            needs_layout_passes=False,
            disable_bounds_checks=True,
        ),
    )(_gather_body)
    return f(xt, table)


def _mask_body(x_ref, o_ref):
    o_ref[...] = x_ref[...] >= 1


@jax.jit
def _tc_mask(x):
    return pl.pallas_call(
        _mask_body,
        out_shape=jax.ShapeDtypeStruct((BATCH, HIST), jnp.bool_),
    )(x)


def kernel(x, weight_matrix):
    l4 = _sc_gather(x.T, weight_matrix)
    embeds = (
        l4.reshape(HIST, 8, NW, 8, BW)
        .transpose(2, 4, 0, 1, 3)
        .reshape(BATCH, HIST, EMBED_DIM)
    )
    mask = _tc_mask(x)
    return embeds, mask
